# baseline (speedup 1.0000x reference)
import functools

import jax
import jax.numpy as jnp
from jax import lax
from jax.experimental import pallas as pl
from jax.experimental.pallas import tpu as pltpu
from jax.experimental.pallas import tpu_sc as plsc

ABLATION = 10000000.0

B = 64
C = 512
HW = 784
G = 56

NW = 32
NC = 2
TC_PLANES = 672
SC_PLANES = HW - TC_PLANES
SC_UNITS = SC_PLANES * 2 // NW


def _reduce_buf(buf, acc):
    def row(b, a):
        for j in range(C // 16):
            a = jnp.minimum(a, buf[b, pl.ds(j * 16, 16)])
        return a

    return lax.fori_loop(0, 32, row, acc)


def _sc_min_body(xt_hbm, out_hbm, buf0, buf1, accbuf, sem0, sem1):
    c = lax.axis_index("c")
    s = lax.axis_index("s")
    w = s * NC + c
    base = w * SC_UNITS + TC_PLANES * 2

    def start(u, buf, sem):
        uu = base + u
        p = uu // 2
        h = uu % 2
        pltpu.make_async_copy(
            xt_hbm.at[p, pl.ds(h * 32, 32), :], buf, sem
        ).start()

    def wait(buf, sem):
        pltpu.make_async_copy(
            xt_hbm.at[0, pl.ds(0, 32), :], buf, sem
        ).wait()

    acc = jnp.full((16,), jnp.inf, jnp.float32)
    start(0, buf0, sem0)
    for t in range(SC_UNITS // 2):
        wait(buf0, sem0)
        start(2 * t + 1, buf1, sem1)
        acc = _reduce_buf(buf0, acc)
        wait(buf1, sem1)
        if 2 * t + 2 < SC_UNITS:
            start(2 * t + 2, buf0, sem0)
        acc = _reduce_buf(buf1, acc)
    accbuf[...] = acc
    pltpu.sync_copy(accbuf, out_hbm.at[w])


@functools.partial(
    pl.kernel,
    out_type=jax.ShapeDtypeStruct((NW, 16), jnp.float32),
    mesh=plsc.VectorSubcoreMesh(core_axis_name="c", subcore_axis_name="s"),
    scratch_types=[
        pltpu.VMEM((32, C), jnp.float32),
        pltpu.VMEM((32, C), jnp.float32),
        pltpu.VMEM((16,), jnp.float32),
        pltpu.SemaphoreType.DMA,
        pltpu.SemaphoreType.DMA,
    ],
)
def _sc_min(xt_hbm, out_hbm, buf0, buf1, accbuf, sem0, sem1):
    _sc_min_body(xt_hbm, out_hbm, buf0, buf1, accbuf, sem0, sem1)


def _tc_min_body(x_ref, out_ref, macc):
    i = pl.program_id(0)
    bmin = jnp.min(x_ref[...])

    @pl.when(i == 0)
    def _():
        macc[0] = bmin

    @pl.when(i > 0)
    def _():
        macc[0] = jnp.minimum(macc[0], bmin)

    @pl.when(i == pl.num_programs(0) - 1)
    def _():
        out_ref[0, 0] = macc[0]


def _tc_min(xt):
    return pl.pallas_call(
        _tc_min_body,
        grid=(TC_PLANES // G,),
        in_specs=[pl.BlockSpec((G, B, C), lambda i: (i, 0, 0))],
        out_specs=pl.BlockSpec(memory_space=pltpu.SMEM),
        out_shape=jax.ShapeDtypeStruct((1, 1), jnp.float32),
        scratch_shapes=[pltpu.SMEM((1,), jnp.float32)],
    )(xt)


def _apply_body(x_ref, mins_ref, mtc_ref, idx_ref, y_ref, vals):
    i = pl.program_id(0)

    @pl.when(i == 0)
    def _():
        m0 = jnp.minimum(jnp.min(mins_ref[...]), mtc_ref[0, 0])
        it = lax.broadcasted_iota(jnp.int32, (B, 1), 0)

        def step(t, s):
            fs = jnp.where(s == 0.0, 0.0, s - ABLATION)
            return jnp.where(it >= t, fs, s)

        vals[...] = lax.fori_loop(0, B, step, jnp.full((B, 1), m0, jnp.float32))

    lane = lax.broadcasted_iota(jnp.int32, (1, B, C), 2)
    eq = lane == idx_ref[...].reshape(1, B, 1)
    vb = jnp.broadcast_to(vals[...].reshape(1, B, 1), (1, B, C))
    y_ref[...] = jnp.where(eq, vb, x_ref[...])


def _apply_pass(xt, mins, mtc, idx):
    return pl.pallas_call(
        _apply_body,
        grid=(HW // G,),
        in_specs=[
            pl.BlockSpec((G, B, C), lambda i: (i, 0, 0)),
            pl.BlockSpec((NW, 16), lambda i: (0, 0)),
            pl.BlockSpec(memory_space=pltpu.SMEM),
            pl.BlockSpec((B, 1), lambda i: (0, 0)),
        ],
        out_specs=pl.BlockSpec((G, B, C), lambda i: (i, 0, 0)),
        out_shape=jax.ShapeDtypeStruct((HW, B, C), jnp.float32),
        scratch_shapes=[pltpu.VMEM((B, 1), jnp.float32)],
    )(xt, mins, mtc, idx)


@jax.jit
def kernel(x, indices):
    xt = x.transpose(2, 3, 0, 1).reshape(HW, B, C)
    mins_sc = _sc_min(xt)
    mtc = _tc_min(xt)
    yt = _apply_pass(xt, mins_sc, mtc, indices.reshape(B, 1))
    return yt.reshape(28, 28, B, C).transpose(2, 3, 0, 1)


# speedup vs baseline: 1.0188x; 1.0188x over previous
import functools

import jax
import jax.numpy as jnp
from jax import lax
from jax.experimental import pallas as pl
from jax.experimental.pallas import tpu as pltpu
from jax.experimental.pallas import tpu_sc as plsc

ABLATION = 10000000.0

B = 64
C = 512
HW = 784
G = 56

NW = 32
NC = 2
TC_PLANES = 560
SC_PLANES = HW - TC_PLANES
SC_UNITS = SC_PLANES * 2 // NW


def _reduce_buf(buf, acc):
    def row(b, a):
        for j in range(C // 16):
            a = jnp.minimum(a, buf[b, pl.ds(j * 16, 16)])
        return a

    return lax.fori_loop(0, 32, row, acc)


def _sc_min_body(xt_hbm, out_hbm, buf0, buf1, accbuf, sem0, sem1):
    c = lax.axis_index("c")
    s = lax.axis_index("s")
    w = s * NC + c
    base = w * SC_UNITS + TC_PLANES * 2

    def start(u, buf, sem):
        uu = base + u
        p = uu // 2
        h = uu % 2
        pltpu.make_async_copy(
            xt_hbm.at[p, pl.ds(h * 32, 32), :], buf, sem
        ).start()

    def wait(buf, sem):
        pltpu.make_async_copy(
            xt_hbm.at[0, pl.ds(0, 32), :], buf, sem
        ).wait()

    acc = jnp.full((16,), jnp.inf, jnp.float32)
    start(0, buf0, sem0)
    for t in range(SC_UNITS // 2):
        wait(buf0, sem0)
        start(2 * t + 1, buf1, sem1)
        acc = _reduce_buf(buf0, acc)
        wait(buf1, sem1)
        if 2 * t + 2 < SC_UNITS:
            start(2 * t + 2, buf0, sem0)
        acc = _reduce_buf(buf1, acc)
    accbuf[...] = acc
    pltpu.sync_copy(accbuf, out_hbm.at[w])


@functools.partial(
    pl.kernel,
    out_type=jax.ShapeDtypeStruct((NW, 16), jnp.float32),
    mesh=plsc.VectorSubcoreMesh(core_axis_name="c", subcore_axis_name="s"),
    scratch_types=[
        pltpu.VMEM((32, C), jnp.float32),
        pltpu.VMEM((32, C), jnp.float32),
        pltpu.VMEM((16,), jnp.float32),
        pltpu.SemaphoreType.DMA,
        pltpu.SemaphoreType.DMA,
    ],
)
def _sc_min(xt_hbm, out_hbm, buf0, buf1, accbuf, sem0, sem1):
    _sc_min_body(xt_hbm, out_hbm, buf0, buf1, accbuf, sem0, sem1)


def _tc_min_body(x_ref, out_ref, macc):
    i = pl.program_id(0)
    bmin = jnp.min(x_ref[...])

    @pl.when(i == 0)
    def _():
        macc[0] = bmin

    @pl.when(i > 0)
    def _():
        macc[0] = jnp.minimum(macc[0], bmin)

    @pl.when(i == pl.num_programs(0) - 1)
    def _():
        out_ref[0, 0] = macc[0]


def _tc_min(xt):
    return pl.pallas_call(
        _tc_min_body,
        grid=(TC_PLANES // G,),
        in_specs=[pl.BlockSpec((G, B, C), lambda i: (i, 0, 0))],
        out_specs=pl.BlockSpec(memory_space=pltpu.SMEM),
        out_shape=jax.ShapeDtypeStruct((1, 1), jnp.float32),
        scratch_shapes=[pltpu.SMEM((1,), jnp.float32)],
    )(xt)


def _apply_body(x_ref, mins_ref, mtc_ref, idx_ref, y_ref, vals):
    i = pl.program_id(0)

    @pl.when(i == 0)
    def _():
        m0 = jnp.minimum(jnp.min(mins_ref[...]), mtc_ref[0, 0])
        it = lax.broadcasted_iota(jnp.int32, (B, 1), 0)

        def step(t, s):
            fs = jnp.where(s == 0.0, 0.0, s - ABLATION)
            return jnp.where(it >= t, fs, s)

        vals[...] = lax.fori_loop(0, B, step, jnp.full((B, 1), m0, jnp.float32))

    lane = lax.broadcasted_iota(jnp.int32, (1, B, C), 2)
    eq = lane == idx_ref[...].reshape(1, B, 1)
    vb = jnp.broadcast_to(vals[...].reshape(1, B, 1), (1, B, C))
    y_ref[...] = jnp.where(eq, vb, x_ref[...])


def _apply_pass(xt, mins, mtc, idx):
    return pl.pallas_call(
        _apply_body,
        grid=(HW // G,),
        in_specs=[
            pl.BlockSpec((G, B, C), lambda i: (i, 0, 0)),
            pl.BlockSpec((NW, 16), lambda i: (0, 0)),
            pl.BlockSpec(memory_space=pltpu.SMEM),
            pl.BlockSpec((B, 1), lambda i: (0, 0)),
        ],
        out_specs=pl.BlockSpec((G, B, C), lambda i: (i, 0, 0)),
        out_shape=jax.ShapeDtypeStruct((HW, B, C), jnp.float32),
        scratch_shapes=[pltpu.VMEM((B, 1), jnp.float32)],
    )(xt, mins, mtc, idx)


@jax.jit
def kernel(x, indices):
    xt = x.transpose(2, 3, 0, 1).reshape(HW, B, C)
    mins_sc = _sc_min(xt)
    mtc = _tc_min(xt)
    yt = _apply_pass(xt, mins_sc, mtc, indices.reshape(B, 1))
    return yt.reshape(28, 28, B, C).transpose(2, 3, 0, 1)
